# baseline (device time: 4231 ns/iter reference)
import jax
import jax.numpy as jnp
from jax import lax
from jax.experimental import pallas as pl
from jax.experimental.pallas import tpu as pltpu

N_DEV = 16
N_CHUNK = 4


def kernel(x):
    m_per, n = x.shape
    total_rows = N_DEV * m_per
    ch = m_per // N_CHUNK

    def body(x_hbm, out_ref, xbuf, copy_sems):
        copies = []

        def start_copy(c):
            dma = pltpu.make_async_copy(
                x_hbm.at[pl.ds(c * ch, ch)], xbuf.at[c % 2], copy_sems.at[c]
            )
            dma.start()
            copies.append(dma)

        start_copy(0)
        acc = None
        for c in range(N_CHUNK):
            if c + 1 < N_CHUNK:
                start_copy(c + 1)
            copies[c].wait()
            s = jnp.sum(xbuf[c % 2], axis=0, keepdims=True)
            acc = s if acc is None else acc + s
        out_ref[...] = acc * (1.0 / total_rows)

    return pl.pallas_call(
        body,
        out_shape=jax.ShapeDtypeStruct((1, n), jnp.float32),
        in_specs=[pl.BlockSpec(memory_space=pl.ANY)],
        out_specs=pl.BlockSpec(memory_space=pltpu.VMEM),
        scratch_shapes=[
            pltpu.VMEM((2, ch, n), jnp.float32),
            pltpu.SemaphoreType.DMA((N_CHUNK,)),
        ],
    )(x)
